# baseline (speedup 1.0000x reference)
import functools

import jax
import jax.numpy as jnp
from jax import lax
from jax.experimental import pallas as pl
from jax.experimental.pallas import tpu as pltpu
from jax.experimental.pallas import tpu_sc as plsc

N = 8192
W3 = 0.25

_info = plsc.get_sparse_core_info()
NC, NS, L = _info.num_cores, _info.num_subcores, _info.num_lanes
NW = NC * NS
PER_W = N // NW
CHUNKS = N // L


def _permute(v, idx):
    return lax.gather(
        v,
        idx.reshape(L, 1),
        lax.GatherDimensionNumbers(
            offset_dims=(), collapsed_slice_dims=(0,), start_index_map=(0,)
        ),
        (1,),
        mode=lax.GatherScatterMode.PROMISE_IN_BOUNDS,
    )


def _splat(v, l):
    return _permute(v, jnp.full((L,), l, jnp.int32))


def _lane_min(v):
    lanes = lax.iota(jnp.int32, L)
    for s in (8, 4, 2, 1):
        v = jnp.minimum(v, _permute(v, lanes ^ s))
    return v


def _sc_kernel(adv_hbm, ori_hbm, out_hbm, adv_v, ori_v, stage_v):
    wid = lax.axis_index("s") * NC + lax.axis_index("c")
    wbase = wid * PER_W
    for k in range(4):
        pltpu.sync_copy(
            adv_hbm.at[k, pl.ds(wbase, PER_W)], adv_v.at[k]
        )
    pltpu.sync_copy(ori_hbm, ori_v)

    def group_body(g, mxs):
        av0 = adv_v[0, pl.ds(g * L, L)]
        av1 = adv_v[1, pl.ds(g * L, L)]
        av2 = adv_v[2, pl.ds(g * L, L)]
        av3 = adv_v[3, pl.ds(g * L, L)]

        def lane_body(l, mxs_in):
            a0 = _splat(av0, l)
            a1 = _splat(av1, l)
            a2 = _splat(av2, l)
            a3 = _splat(av3, l)

            def ori_body(c, m16):
                o0 = ori_v[0, pl.ds(c * L, L)]
                o1 = ori_v[1, pl.ds(c * L, L)]
                o2 = ori_v[2, pl.ds(c * L, L)]
                o3 = ori_v[3, pl.ds(c * L, L)]
                d0 = o0 - a0
                d1 = o1 - a1
                d2 = o2 - a2
                d3 = o3 - a3
                acc = d0 * d0 + d1 * d1 + d2 * d2 + W3 * (d3 * d3)
                return jnp.minimum(m16, acc)

            m16 = lax.fori_loop(
                0, CHUNKS, ori_body, jnp.full((L,), jnp.inf, jnp.float32)
            )
            return jnp.maximum(mxs_in, _lane_min(m16))

        return lax.fori_loop(0, L, lane_body, mxs)

    mx16 = lax.fori_loop(
        0, PER_W // L, group_body, jnp.full((L,), -jnp.inf, jnp.float32)
    )
    stage_v[...] = mx16
    pltpu.sync_copy(stage_v, out_hbm.at[wid])


def kernel(adv_pc, ori_pc):
    adv_t = adv_pc.T
    ori_t = ori_pc.T
    mesh = plsc.VectorSubcoreMesh(core_axis_name="c", subcore_axis_name="s")
    k = functools.partial(
        pl.kernel,
        mesh=mesh,
        out_type=jax.ShapeDtypeStruct((NW, L), jnp.float32),
        scratch_types=[
            pltpu.VMEM((4, PER_W), jnp.float32),
            pltpu.VMEM((4, N), jnp.float32),
            pltpu.VMEM((L,), jnp.float32),
        ],
    )(_sc_kernel)
    partials = k(adv_t, ori_t)
    return jnp.max(partials).reshape(1)


# speedup vs baseline: 15.7478x; 15.7478x over previous
import jax
import jax.numpy as jnp
from jax.experimental import pallas as pl
from jax.experimental.pallas import tpu as pltpu

N = 8192
BLK = 256
K = 24
W3 = 0.25


def _split(x):
    hi = x.astype(jnp.bfloat16)
    lo = (x - hi.astype(jnp.float32)).astype(jnp.bfloat16)
    return hi, lo


def _hd_body(adv_t_ref, ori_t_ref, out_ref, aaug_ref, baug_ref):
    a0 = adv_t_ref[0:1, :]
    a1 = adv_t_ref[1:2, :]
    a2 = adv_t_ref[2:3, :]
    a3 = adv_t_ref[3:4, :]
    na = a0 * a0 + a1 * a1 + a2 * a2 + W3 * (a3 * a3)
    ah0, al0 = _split(a0)
    ah1, al1 = _split(a1)
    ah2, al2 = _split(a2)
    ah3, al3 = _split(a3)
    nah, nal = _split(na)
    ones_r = jnp.ones((1, N), jnp.bfloat16)
    aaug_ref[...] = jnp.concatenate(
        [
            ah0, ah1, ah2, ah3,
            ah0, ah1, ah2, ah3,
            al0, al1, al2, al3,
            al0, al1, al2, al3,
            ones_r, ones_r, nah, nal,
            jnp.zeros((K - 20, N), jnp.bfloat16),
        ],
        axis=0,
    )

    b0 = ori_t_ref[0:1, :]
    b1 = ori_t_ref[1:2, :]
    b2 = ori_t_ref[2:3, :]
    b3 = ori_t_ref[3:4, :]
    nb = b0 * b0 + b1 * b1 + b2 * b2 + W3 * (b3 * b3)
    bh0, bl0 = _split(-2.0 * b0)
    bh1, bl1 = _split(-2.0 * b1)
    bh2, bl2 = _split(-2.0 * b2)
    bh3, bl3 = _split(-0.5 * b3)
    nbh, nbl = _split(nb)
    baug_ref[...] = jnp.concatenate(
        [
            bh0, bh1, bh2, bh3,
            bl0, bl1, bl2, bl3,
            bh0, bh1, bh2, bh3,
            bl0, bl1, bl2, bl3,
            nbh, nbl, ones_r, ones_r,
            jnp.zeros((K - 20, N), jnp.bfloat16),
        ],
        axis=0,
    )

    def _tile_min(j):
        b_blk = baug_ref[:, pl.ds(j * BLK, BLK)]
        e = jax.lax.dot_general(
            b_blk,
            aaug_ref[...],
            (((0,), (0,)), ((), ())),
            preferred_element_type=jnp.float32,
        )
        return jnp.min(e.reshape(BLK // 8, 8, N), axis=0)

    def body(jj, m):
        e8a = jnp.minimum(_tile_min(4 * jj), _tile_min(4 * jj + 1))
        e8b = jnp.minimum(_tile_min(4 * jj + 2), _tile_min(4 * jj + 3))
        return jnp.minimum(m, jnp.minimum(e8a, e8b))

    m = jax.lax.fori_loop(
        0, N // (4 * BLK), body, jnp.full((8, N), jnp.inf, jnp.float32)
    )

    nn = jnp.min(m, axis=0)
    out_ref[...] = jnp.max(nn).reshape(1, 1)


def kernel(adv_pc, ori_pc):
    adv_t = adv_pc.T
    ori_t = ori_pc.T
    out = pl.pallas_call(
        _hd_body,
        out_shape=jax.ShapeDtypeStruct((1, 1), jnp.float32),
        scratch_shapes=[
            pltpu.VMEM((K, N), jnp.bfloat16),
            pltpu.VMEM((K, N), jnp.bfloat16),
        ],
    )(adv_t, ori_t)
    return out.reshape(1)


# speedup vs baseline: 16.1316x; 1.0244x over previous
import jax
import jax.numpy as jnp
from jax.experimental import pallas as pl
from jax.experimental.pallas import tpu as pltpu

N = 8192
BLK = 512
K = 24
W3 = 0.25


def _split(x):
    hi = x.astype(jnp.bfloat16)
    lo = (x - hi.astype(jnp.float32)).astype(jnp.bfloat16)
    return hi, lo


def _hd_body(adv_t_ref, ori_t_ref, out_ref, aaug_ref, baug_ref):
    a0 = adv_t_ref[0:1, :]
    a1 = adv_t_ref[1:2, :]
    a2 = adv_t_ref[2:3, :]
    a3 = adv_t_ref[3:4, :]
    na = a0 * a0 + a1 * a1 + a2 * a2 + W3 * (a3 * a3)
    ah0, al0 = _split(a0)
    ah1, al1 = _split(a1)
    ah2, al2 = _split(a2)
    ah3, al3 = _split(a3)
    nah, nal = _split(na)
    ones_r = jnp.ones((1, N), jnp.bfloat16)
    aaug_ref[...] = jnp.concatenate(
        [
            ah0, ah1, ah2, ah3,
            ah0, ah1, ah2, ah3,
            al0, al1, al2, al3,
            al0, al1, al2, al3,
            ones_r, ones_r, nah, nal,
            jnp.zeros((K - 20, N), jnp.bfloat16),
        ],
        axis=0,
    )

    b0 = ori_t_ref[0:1, :]
    b1 = ori_t_ref[1:2, :]
    b2 = ori_t_ref[2:3, :]
    b3 = ori_t_ref[3:4, :]
    nb = b0 * b0 + b1 * b1 + b2 * b2 + W3 * (b3 * b3)
    bh0, bl0 = _split(-2.0 * b0)
    bh1, bl1 = _split(-2.0 * b1)
    bh2, bl2 = _split(-2.0 * b2)
    bh3, bl3 = _split(-0.5 * b3)
    nbh, nbl = _split(nb)
    baug_ref[...] = jnp.concatenate(
        [
            bh0, bh1, bh2, bh3,
            bl0, bl1, bl2, bl3,
            bh0, bh1, bh2, bh3,
            bl0, bl1, bl2, bl3,
            nbh, nbl, ones_r, ones_r,
            jnp.zeros((K - 20, N), jnp.bfloat16),
        ],
        axis=0,
    )

    def _tile_min(j):
        b_blk = baug_ref[:, pl.ds(j * BLK, BLK)]
        e = jax.lax.dot_general(
            b_blk,
            aaug_ref[...],
            (((0,), (0,)), ((), ())),
            preferred_element_type=jnp.float32,
        )
        return jnp.min(e.reshape(BLK // 8, 8, N), axis=0)

    def body(jj, m):
        e8a = jnp.minimum(_tile_min(4 * jj), _tile_min(4 * jj + 1))
        e8b = jnp.minimum(_tile_min(4 * jj + 2), _tile_min(4 * jj + 3))
        return jnp.minimum(m, jnp.minimum(e8a, e8b))

    m = jax.lax.fori_loop(
        0, N // (4 * BLK), body, jnp.full((8, N), jnp.inf, jnp.float32)
    )

    nn = jnp.min(m, axis=0)
    out_ref[...] = jnp.max(nn).reshape(1, 1)


def kernel(adv_pc, ori_pc):
    adv_t = adv_pc.T
    ori_t = ori_pc.T
    out = pl.pallas_call(
        _hd_body,
        out_shape=jax.ShapeDtypeStruct((1, 1), jnp.float32),
        scratch_shapes=[
            pltpu.VMEM((K, N), jnp.bfloat16),
            pltpu.VMEM((K, N), jnp.bfloat16),
        ],
    )(adv_t, ori_t)
    return out.reshape(1)


# speedup vs baseline: 16.3194x; 1.0116x over previous
import jax
import jax.numpy as jnp
from jax.experimental import pallas as pl
from jax.experimental.pallas import tpu as pltpu

N = 8192
BLK = 1024
K = 24
W3 = 0.25


def _split(x):
    hi = x.astype(jnp.bfloat16)
    lo = (x - hi.astype(jnp.float32)).astype(jnp.bfloat16)
    return hi, lo


def _hd_body(adv_t_ref, ori_t_ref, out_ref, aaug_ref, baug_ref):
    a0 = adv_t_ref[0:1, :]
    a1 = adv_t_ref[1:2, :]
    a2 = adv_t_ref[2:3, :]
    a3 = adv_t_ref[3:4, :]
    na = a0 * a0 + a1 * a1 + a2 * a2 + W3 * (a3 * a3)
    ah0, al0 = _split(a0)
    ah1, al1 = _split(a1)
    ah2, al2 = _split(a2)
    ah3, al3 = _split(a3)
    nah, nal = _split(na)
    ones_r = jnp.ones((1, N), jnp.bfloat16)
    aaug_ref[...] = jnp.concatenate(
        [
            ah0, ah1, ah2, ah3,
            ah0, ah1, ah2, ah3,
            al0, al1, al2, al3,
            al0, al1, al2, al3,
            ones_r, ones_r, nah, nal,
            jnp.zeros((K - 20, N), jnp.bfloat16),
        ],
        axis=0,
    )

    b0 = ori_t_ref[0:1, :]
    b1 = ori_t_ref[1:2, :]
    b2 = ori_t_ref[2:3, :]
    b3 = ori_t_ref[3:4, :]
    nb = b0 * b0 + b1 * b1 + b2 * b2 + W3 * (b3 * b3)
    bh0, bl0 = _split(-2.0 * b0)
    bh1, bl1 = _split(-2.0 * b1)
    bh2, bl2 = _split(-2.0 * b2)
    bh3, bl3 = _split(-0.5 * b3)
    nbh, nbl = _split(nb)
    baug_ref[...] = jnp.concatenate(
        [
            bh0, bh1, bh2, bh3,
            bl0, bl1, bl2, bl3,
            bh0, bh1, bh2, bh3,
            bl0, bl1, bl2, bl3,
            nbh, nbl, ones_r, ones_r,
            jnp.zeros((K - 20, N), jnp.bfloat16),
        ],
        axis=0,
    )

    def _tile_min(j):
        b_blk = baug_ref[:, pl.ds(j * BLK, BLK)]
        e = jax.lax.dot_general(
            b_blk,
            aaug_ref[...],
            (((0,), (0,)), ((), ())),
            preferred_element_type=jnp.float32,
        )
        return jnp.min(e.reshape(BLK // 8, 8, N), axis=0)

    def body(jj, m):
        e8a = jnp.minimum(_tile_min(4 * jj), _tile_min(4 * jj + 1))
        e8b = jnp.minimum(_tile_min(4 * jj + 2), _tile_min(4 * jj + 3))
        return jnp.minimum(m, jnp.minimum(e8a, e8b))

    m = jax.lax.fori_loop(
        0, N // (4 * BLK), body, jnp.full((8, N), jnp.inf, jnp.float32)
    )

    nn = jnp.min(m, axis=0)
    out_ref[...] = jnp.max(nn).reshape(1, 1)


def kernel(adv_pc, ori_pc):
    adv_t = adv_pc.T
    ori_t = ori_pc.T
    out = pl.pallas_call(
        _hd_body,
        out_shape=jax.ShapeDtypeStruct((1, 1), jnp.float32),
        scratch_shapes=[
            pltpu.VMEM((K, N), jnp.bfloat16),
            pltpu.VMEM((K, N), jnp.bfloat16),
        ],
    )(adv_t, ori_t)
    return out.reshape(1)


# speedup vs baseline: 16.6790x; 1.0220x over previous
import jax
import jax.numpy as jnp
from jax.experimental import pallas as pl
from jax.experimental.pallas import tpu as pltpu

N = 8192
BLK = 2048
K = 24
W3 = 0.25


def _split(x):
    hi = x.astype(jnp.bfloat16)
    lo = (x - hi.astype(jnp.float32)).astype(jnp.bfloat16)
    return hi, lo


def _hd_body(adv_t_ref, ori_t_ref, out_ref, aaug_ref, baug_ref):
    a0 = adv_t_ref[0:1, :]
    a1 = adv_t_ref[1:2, :]
    a2 = adv_t_ref[2:3, :]
    a3 = adv_t_ref[3:4, :]
    na = a0 * a0 + a1 * a1 + a2 * a2 + W3 * (a3 * a3)
    ah0, al0 = _split(a0)
    ah1, al1 = _split(a1)
    ah2, al2 = _split(a2)
    ah3, al3 = _split(a3)
    nah, nal = _split(na)
    ones_r = jnp.ones((1, N), jnp.bfloat16)
    aaug_ref[...] = jnp.concatenate(
        [
            ah0, ah1, ah2, ah3,
            ah0, ah1, ah2, ah3,
            al0, al1, al2, al3,
            al0, al1, al2, al3,
            ones_r, ones_r, nah, nal,
            jnp.zeros((K - 20, N), jnp.bfloat16),
        ],
        axis=0,
    )

    b0 = ori_t_ref[0:1, :]
    b1 = ori_t_ref[1:2, :]
    b2 = ori_t_ref[2:3, :]
    b3 = ori_t_ref[3:4, :]
    nb = b0 * b0 + b1 * b1 + b2 * b2 + W3 * (b3 * b3)
    bh0, bl0 = _split(-2.0 * b0)
    bh1, bl1 = _split(-2.0 * b1)
    bh2, bl2 = _split(-2.0 * b2)
    bh3, bl3 = _split(-0.5 * b3)
    nbh, nbl = _split(nb)
    baug_ref[...] = jnp.concatenate(
        [
            bh0, bh1, bh2, bh3,
            bl0, bl1, bl2, bl3,
            bh0, bh1, bh2, bh3,
            bl0, bl1, bl2, bl3,
            nbh, nbl, ones_r, ones_r,
            jnp.zeros((K - 20, N), jnp.bfloat16),
        ],
        axis=0,
    )

    def _tile_min(j):
        b_blk = baug_ref[:, pl.ds(j * BLK, BLK)]
        e = jax.lax.dot_general(
            b_blk,
            aaug_ref[...],
            (((0,), (0,)), ((), ())),
            preferred_element_type=jnp.float32,
        )
        return jnp.min(e.reshape(BLK // 8, 8, N), axis=0)

    def body(jj, m):
        e8a = jnp.minimum(_tile_min(4 * jj), _tile_min(4 * jj + 1))
        e8b = jnp.minimum(_tile_min(4 * jj + 2), _tile_min(4 * jj + 3))
        return jnp.minimum(m, jnp.minimum(e8a, e8b))

    m = jax.lax.fori_loop(
        0, N // (4 * BLK), body, jnp.full((8, N), jnp.inf, jnp.float32)
    )

    nn = jnp.min(m, axis=0)
    out_ref[...] = jnp.max(nn).reshape(1, 1)


def kernel(adv_pc, ori_pc):
    adv_t = adv_pc.T
    ori_t = ori_pc.T
    out = pl.pallas_call(
        _hd_body,
        out_shape=jax.ShapeDtypeStruct((1, 1), jnp.float32),
        scratch_shapes=[
            pltpu.VMEM((K, N), jnp.bfloat16),
            pltpu.VMEM((K, N), jnp.bfloat16),
        ],
    )(adv_t, ori_t)
    return out.reshape(1)


# speedup vs baseline: 16.7473x; 1.0041x over previous
import jax
import jax.numpy as jnp
from jax.experimental import pallas as pl
from jax.experimental.pallas import tpu as pltpu

N = 8192
BLK = 8192
K = 24
W3 = 0.25


def _split(x):
    hi = x.astype(jnp.bfloat16)
    lo = (x - hi.astype(jnp.float32)).astype(jnp.bfloat16)
    return hi, lo


def _hd_body(adv_t_ref, ori_t_ref, out_ref, aaug_ref, baug_ref):
    a0 = adv_t_ref[0:1, :]
    a1 = adv_t_ref[1:2, :]
    a2 = adv_t_ref[2:3, :]
    a3 = adv_t_ref[3:4, :]
    na = a0 * a0 + a1 * a1 + a2 * a2 + W3 * (a3 * a3)
    ah0, al0 = _split(a0)
    ah1, al1 = _split(a1)
    ah2, al2 = _split(a2)
    ah3, al3 = _split(a3)
    nah, nal = _split(na)
    ones_r = jnp.ones((1, N), jnp.bfloat16)
    aaug_ref[...] = jnp.concatenate(
        [
            ah0, ah1, ah2, ah3,
            ah0, ah1, ah2, ah3,
            al0, al1, al2, al3,
            al0, al1, al2, al3,
            ones_r, ones_r, nah, nal,
            jnp.zeros((K - 20, N), jnp.bfloat16),
        ],
        axis=0,
    )

    b0 = ori_t_ref[0:1, :]
    b1 = ori_t_ref[1:2, :]
    b2 = ori_t_ref[2:3, :]
    b3 = ori_t_ref[3:4, :]
    nb = b0 * b0 + b1 * b1 + b2 * b2 + W3 * (b3 * b3)
    bh0, bl0 = _split(-2.0 * b0)
    bh1, bl1 = _split(-2.0 * b1)
    bh2, bl2 = _split(-2.0 * b2)
    bh3, bl3 = _split(-0.5 * b3)
    nbh, nbl = _split(nb)
    baug_ref[...] = jnp.concatenate(
        [
            bh0, bh1, bh2, bh3,
            bl0, bl1, bl2, bl3,
            bh0, bh1, bh2, bh3,
            bl0, bl1, bl2, bl3,
            nbh, nbl, ones_r, ones_r,
            jnp.zeros((K - 20, N), jnp.bfloat16),
        ],
        axis=0,
    )

    def _tile_min(j):
        b_blk = baug_ref[:, pl.ds(j * BLK, BLK)]
        e = jax.lax.dot_general(
            b_blk,
            aaug_ref[...],
            (((0,), (0,)), ((), ())),
            preferred_element_type=jnp.float32,
        )
        return jnp.min(e.reshape(BLK // 8, 8, N), axis=0)

    tiles = [_tile_min(t) for t in range(N // BLK)]
    m = tiles[0]
    for t in tiles[1:]:
        m = jnp.minimum(m, t)

    nn = jnp.min(m, axis=0)
    out_ref[...] = jnp.max(nn).reshape(1, 1)


def kernel(adv_pc, ori_pc):
    adv_t = adv_pc.T
    ori_t = ori_pc.T
    out = pl.pallas_call(
        _hd_body,
        out_shape=jax.ShapeDtypeStruct((1, 1), jnp.float32),
        scratch_shapes=[
            pltpu.VMEM((K, N), jnp.bfloat16),
            pltpu.VMEM((K, N), jnp.bfloat16),
        ],
    )(adv_t, ori_t)
    return out.reshape(1)
